# Initial kernel scaffold; baseline (speedup 1.0000x reference)
import jax
import jax.numpy as jnp
from jax.experimental import pallas as pl


def kernel(x, edge_index, edge_attr, batch, W_l, W_r, att, bias, W_res):
    raise NotImplementedError("write your pallas kernel here")


# speedup vs baseline: 12.7410x; 12.7410x over previous
import functools

import jax
import jax.numpy as jnp
from jax import lax
from jax.experimental import pallas as pl
from jax.experimental.pallas import tpu as pltpu
from jax.experimental.pallas import tpu_sc as plsc

F32 = jnp.float32
I32 = jnp.int32

NC = 2
NS = 16
NW = NC * NS
L = 16


def _matmuls(x, w_lc, w_rc, w_lh, w_res):
    n, k = x.shape
    hc = w_lc.shape[1]
    oc = w_res.shape[1]
    mb = 1000
    grid = n // mb

    def body(x_ref, wlc_ref, wrc_ref, wlh_ref, wres_ref, xlc_ref, xrc_ref,
             xlh_ref, res_ref):
        xb = x_ref[...]
        xlc_ref[...] = jnp.dot(xb, wlc_ref[...], preferred_element_type=F32)
        xrc_ref[...] = jnp.dot(xb, wrc_ref[...], preferred_element_type=F32)
        xlh_ref[...] = jnp.dot(xb, wlh_ref[...], preferred_element_type=F32)
        res_ref[...] = jnp.dot(xb, wres_ref[...], preferred_element_type=F32)

    return pl.pallas_call(
        body,
        grid=(grid,),
        in_specs=[
            pl.BlockSpec((mb, k), lambda i: (i, 0)),
            pl.BlockSpec((k, hc), lambda i: (0, 0)),
            pl.BlockSpec((k, hc), lambda i: (0, 0)),
            pl.BlockSpec((k, hc), lambda i: (0, 0)),
            pl.BlockSpec((k, oc), lambda i: (0, 0)),
        ],
        out_specs=[
            pl.BlockSpec((mb, hc), lambda i: (i, 0)),
            pl.BlockSpec((mb, hc), lambda i: (i, 0)),
            pl.BlockSpec((mb, hc), lambda i: (i, 0)),
            pl.BlockSpec((mb, oc), lambda i: (i, 0)),
        ],
        out_shape=[
            jax.ShapeDtypeStruct((n, hc), F32),
            jax.ShapeDtypeStruct((n, hc), F32),
            jax.ShapeDtypeStruct((n, hc), F32),
            jax.ShapeDtypeStruct((n, oc), F32),
        ],
    )(x, w_lc, w_rc, w_lh, w_res)


def _pass1(xlc, xrc, src, dst, att_t, n, e, h, c, b, chunks):
    np_ = ((n + 8 * NS - 1) // (8 * NS)) * (8 * NS)
    rows_sub = np_ // NS
    mesh = plsc.VectorSubcoreMesh(core_axis_name="c", subcore_axis_name="s")
    epw = e // NW

    @functools.partial(
        pl.kernel,
        out_type=[
            jax.ShapeDtypeStruct((e, h), F32),
            jax.ShapeDtypeStruct((np_, h), F32),
            jax.ShapeDtypeStruct((np_, h), F32),
        ],
        mesh=mesh,
        compiler_params=pltpu.CompilerParams(use_tc_tiling_on_sc=False),
        scratch_types=[
            pltpu.VMEM((b,), I32),
            pltpu.VMEM((b,), I32),
            pltpu.VMEM((b, h * c), F32),
            pltpu.VMEM((b, h * c), F32),
            pltpu.VMEM((b, h), F32),
            pltpu.VMEM((c, h), F32),
            pltpu.VMEM((rows_sub, h), F32),
            pltpu.VMEM_SHARED((np_, h), F32),
            pltpu.SemaphoreType.DMA,
            pltpu.SemaphoreType.DMA,
        ],
    )
    def p1(xlc_hbm, xrc_hbm, src_hbm, dst_hbm, att_hbm,
           expv_hbm, den0_hbm, den1_hbm,
           idx_s, idx_d, xlbuf, xrbuf, expbuf, attv, zbuf, den_sh,
           sem1, sem2):
        core = lax.axis_index("c")
        sub = lax.axis_index("s")
        wid = core * NS + sub

        pltpu.sync_copy(att_hbm, attv)

        zero = jnp.zeros((L,), F32)

        def zrow(i, carry):
            zbuf[i, :] = zero
            return carry

        lax.fori_loop(0, rows_sub, zrow, 0)
        pltpu.sync_copy(zbuf, den_sh.at[pl.ds(sub * rows_sub, rows_sub)])
        plsc.subcore_barrier()

        def chunk(k, carry):
            base = wid * epw + k * b
            pltpu.sync_copy(src_hbm.at[pl.ds(base, b)], idx_s)
            pltpu.sync_copy(dst_hbm.at[pl.ds(base, b)], idx_d)
            pltpu.async_copy(xlc_hbm.at[idx_s], xlbuf, sem1).wait()
            pltpu.async_copy(xrc_hbm.at[idx_d], xrbuf, sem2).wait()

            def edge(ei, icarry):
                acc = jnp.zeros((L,), F32)
                for ci in range(c):
                    u = (xlbuf[ei, pl.ds(ci * L, L)]
                         + xrbuf[ei, pl.ds(ci * L, L)])
                    u = jnp.maximum(u, 0.2 * u)
                    acc = acc + u * attv[ci, :]
                expbuf[ei, :] = jnp.exp(acc)
                return icarry

            lax.fori_loop(0, b, edge, 0)
            pltpu.sync_copy(expbuf, expv_hbm.at[pl.ds(base, b)])
            pltpu.sync_copy(expbuf, den_sh.at[idx_d], add=True)
            return carry

        lax.fori_loop(0, chunks, chunk, 0)
        plsc.subcore_barrier()

        rows = pl.ds(sub * rows_sub, rows_sub)

        @pl.when(core == 0)
        def _():
            pltpu.sync_copy(den_sh.at[rows], den0_hbm.at[rows])

        @pl.when(core == 1)
        def _():
            pltpu.sync_copy(den_sh.at[rows], den1_hbm.at[rows])

    return p1(xlc, xrc, src, dst, att_t)


def _pass2(xlh, src, dst, expv, den0, den1, n, e, h, c, b, chunks):
    np_ = ((n + 8 * NS - 1) // (8 * NS)) * (8 * NS)
    rows_sub = np_ // NS
    mesh = plsc.VectorSubcoreMesh(core_axis_name="c", subcore_axis_name="s")
    epw = e // NW

    @functools.partial(
        pl.kernel,
        out_type=[
            jax.ShapeDtypeStruct((e, h), F32),
            jax.ShapeDtypeStruct((np_, c), F32),
            jax.ShapeDtypeStruct((np_, c), F32),
        ],
        mesh=mesh,
        compiler_params=pltpu.CompilerParams(use_tc_tiling_on_sc=False),
        scratch_types=[
            pltpu.VMEM((b,), I32),
            pltpu.VMEM((b,), I32),
            pltpu.VMEM((b, h * c), F32),
            pltpu.VMEM((b, h), F32),
            pltpu.VMEM((b, h), F32),
            pltpu.VMEM((b, h), F32),
            pltpu.VMEM((b, h), F32),
            pltpu.VMEM((b, c), F32),
            pltpu.VMEM((rows_sub, c), F32),
            pltpu.VMEM_SHARED((np_, c), F32),
            pltpu.SemaphoreType.DMA,
            pltpu.SemaphoreType.DMA,
            pltpu.SemaphoreType.DMA,
        ],
    )
    def p2(xlh_hbm, src_hbm, dst_hbm, expv_hbm, den0_hbm, den1_hbm,
           alpha_hbm, t0_hbm, t1_hbm,
           idx_s, idx_d, xlbuf, expbuf, d0buf, d1buf, albuf, msgbuf, zbuf,
           t_sh, sem1, sem2, sem3):
        core = lax.axis_index("c")
        sub = lax.axis_index("s")
        wid = core * NS + sub

        zero = jnp.zeros((L,), F32)

        def zrow(i, carry):
            zbuf[i, pl.ds(0, L)] = zero
            zbuf[i, pl.ds(L, L)] = zero
            return carry

        lax.fori_loop(0, rows_sub, zrow, 0)
        pltpu.sync_copy(zbuf, t_sh.at[pl.ds(sub * rows_sub, rows_sub)])
        plsc.subcore_barrier()

        def chunk(k, carry):
            base = wid * epw + k * b
            pltpu.sync_copy(src_hbm.at[pl.ds(base, b)], idx_s)
            pltpu.sync_copy(dst_hbm.at[pl.ds(base, b)], idx_d)
            pltpu.sync_copy(expv_hbm.at[pl.ds(base, b)], expbuf)
            pltpu.async_copy(xlh_hbm.at[idx_s], xlbuf, sem1).wait()
            pltpu.async_copy(den0_hbm.at[idx_d], d0buf, sem2).wait()
            pltpu.async_copy(den1_hbm.at[idx_d], d1buf, sem3).wait()

            def edge(ei, icarry):
                den = d0buf[ei, :] + d1buf[ei, :]
                alpha_v = expbuf[ei, :] / (den + 1e-16)
                albuf[ei, :] = alpha_v
                for cg in range(c // L):
                    m = jnp.zeros((L,), F32)
                    for hi in range(h):
                        a_s = alpha_v[hi]
                        m = m + a_s * xlbuf[ei, pl.ds(hi * c + cg * L, L)]
                    msgbuf[ei, pl.ds(cg * L, L)] = m
                return icarry

            lax.fori_loop(0, b, edge, 0)
            pltpu.sync_copy(albuf, alpha_hbm.at[pl.ds(base, b)])
            pltpu.sync_copy(msgbuf, t_sh.at[idx_d], add=True)
            return carry

        lax.fori_loop(0, chunks, chunk, 0)
        plsc.subcore_barrier()

        rows = pl.ds(sub * rows_sub, rows_sub)

        @pl.when(core == 0)
        def _():
            pltpu.sync_copy(t_sh.at[rows], t0_hbm.at[rows])

        @pl.when(core == 1)
        def _():
            pltpu.sync_copy(t_sh.at[rows], t1_hbm.at[rows])

    return p2(xlh, src, dst, expv, den0, den1)


def _finalize(t0, t1, res, bias, h):
    n, oc = res.shape
    mb = 1000
    inv_h = 1.0 / h

    def body(t0_ref, t1_ref, res_ref, bias_ref, out_ref):
        g = (t0_ref[...] + t1_ref[...]) * inv_h + bias_ref[...]
        out_ref[...] = jnp.maximum(g, 0.0) + res_ref[...]

    return pl.pallas_call(
        body,
        grid=(n // mb,),
        in_specs=[
            pl.BlockSpec((mb, oc), lambda i: (i, 0)),
            pl.BlockSpec((mb, oc), lambda i: (i, 0)),
            pl.BlockSpec((mb, oc), lambda i: (i, 0)),
            pl.BlockSpec((1, oc), lambda i: (0, 0)),
        ],
        out_specs=pl.BlockSpec((mb, oc), lambda i: (i, 0)),
        out_shape=jax.ShapeDtypeStruct((n, oc), F32),
    )(t0, t1, res, bias.reshape(1, oc))


def kernel(x, edge_index, edge_attr, batch, W_l, W_r, att, bias, W_res):
    n, in_ch = x.shape
    h, c = att.shape
    e = edge_index.shape[1]
    b = 40
    chunks = e // (NW * b)

    src = edge_index[0].astype(I32)
    dst = edge_index[1].astype(I32)

    w_lc = W_l.reshape(in_ch, h, c).transpose(0, 2, 1).reshape(in_ch, h * c)
    w_rc = W_r.reshape(in_ch, h, c).transpose(0, 2, 1).reshape(in_ch, h * c)
    att_t = att.T.astype(F32)

    xlc, xrc, xlh, res = _matmuls(x, w_lc, w_rc, W_l, W_res)
    expv, den0, den1 = _pass1(xlc, xrc, src, dst, att_t, n, e, h, c, b, chunks)
    alpha, t0, t1 = _pass2(xlh, src, dst, expv, den0, den1, n, e, h, c, b,
                           chunks)
    x_out = _finalize(t0[:n], t1[:n], res, bias, h)
    return (x_out, edge_index, edge_attr, batch, alpha)


# speedup vs baseline: 26.4408x; 2.0752x over previous
import functools

import jax
import jax.numpy as jnp
from jax import lax
from jax.experimental import pallas as pl
from jax.experimental.pallas import tpu as pltpu
from jax.experimental.pallas import tpu_sc as plsc

F32 = jnp.float32
I32 = jnp.int32

NC = 2
NS = 16
NW = NC * NS
L = 16
ZR = 64


def _matmuls(x, w_lc, w_rc, w_lh, w_res):
    n, k = x.shape
    hc = w_lc.shape[1]
    oc = w_res.shape[1]
    mb = 1000
    grid = n // mb

    def body(x_ref, wlc_ref, wrc_ref, wlh_ref, wres_ref, xlc_ref, xrc_ref,
             xlh_ref, res_ref):
        xb = x_ref[...]
        xlc_ref[...] = jnp.dot(xb, wlc_ref[...], preferred_element_type=F32)
        xrc_ref[...] = jnp.dot(xb, wrc_ref[...], preferred_element_type=F32)
        xlh_ref[...] = jnp.dot(xb, wlh_ref[...], preferred_element_type=F32)
        res_ref[...] = jnp.dot(xb, wres_ref[...], preferred_element_type=F32)

    return pl.pallas_call(
        body,
        grid=(grid,),
        in_specs=[
            pl.BlockSpec((mb, k), lambda i: (i, 0)),
            pl.BlockSpec((k, hc), lambda i: (0, 0)),
            pl.BlockSpec((k, hc), lambda i: (0, 0)),
            pl.BlockSpec((k, hc), lambda i: (0, 0)),
            pl.BlockSpec((k, oc), lambda i: (0, 0)),
        ],
        out_specs=[
            pl.BlockSpec((mb, hc), lambda i: (i, 0)),
            pl.BlockSpec((mb, hc), lambda i: (i, 0)),
            pl.BlockSpec((mb, hc), lambda i: (i, 0)),
            pl.BlockSpec((mb, oc), lambda i: (i, 0)),
        ],
        out_shape=[
            jax.ShapeDtypeStruct((n, hc), F32),
            jax.ShapeDtypeStruct((n, hc), F32),
            jax.ShapeDtypeStruct((n, hc), F32),
            jax.ShapeDtypeStruct((n, oc), F32),
        ],
    )(x, w_lc, w_rc, w_lh, w_res)


def _zero_shared(zbuf, shared, sub, rows_sub, width):
    zero = jnp.zeros((L,), F32)

    def zrow(i, carry):
        for g in range(width // L):
            zbuf[i, pl.ds(g * L, L)] = zero
        return carry

    lax.fori_loop(0, rows_sub, zrow, 0)
    pltpu.sync_copy(zbuf, shared.at[pl.ds(sub * rows_sub, rows_sub)])


def _pass1(xlc, xrc, src3, dst3, att_t, n, e, h, c, b, chunks):
    np_ = ((n + 8 * NS - 1) // (8 * NS)) * (8 * NS)
    rows_sub = np_ // NS
    mesh = plsc.VectorSubcoreMesh(core_axis_name="c", subcore_axis_name="s")
    epw = e // NW
    hc = h * c
    pairs = (chunks - 1) // 2

    @functools.partial(
        pl.kernel,
        out_type=[
            jax.ShapeDtypeStruct((e, h), F32),
            jax.ShapeDtypeStruct((np_, h), F32),
            jax.ShapeDtypeStruct((np_, h), F32),
        ],
        mesh=mesh,
        compiler_params=pltpu.CompilerParams(use_tc_tiling_on_sc=False),
        scratch_types=[
            pltpu.VMEM((chunks, b), I32),
            pltpu.VMEM((chunks, b), I32),
            pltpu.VMEM((b, hc), F32),
            pltpu.VMEM((b, hc), F32),
            pltpu.VMEM((b, hc), F32),
            pltpu.VMEM((b, hc), F32),
            pltpu.VMEM((b, h), F32),
            pltpu.VMEM((c, h), F32),
            pltpu.VMEM((rows_sub, h), F32),
            pltpu.VMEM_SHARED((np_, h), F32),
            pltpu.SemaphoreType.DMA,
            pltpu.SemaphoreType.DMA,
            pltpu.SemaphoreType.DMA,
            pltpu.SemaphoreType.DMA,
        ],
    )
    def p1(xlc_hbm, xrc_hbm, src_hbm, dst_hbm, att_hbm,
           expv_hbm, den0_hbm, den1_hbm,
           src_sl, dst_sl, xl0, xl1, xr0, xr1, expbuf, attv, zbuf, den_sh,
           sl0, sl1, sr0, sr1):
        core = lax.axis_index("c")
        sub = lax.axis_index("s")
        wid = core * NS + sub

        pltpu.sync_copy(att_hbm, attv)
        pltpu.sync_copy(src_hbm.at[wid], src_sl)
        pltpu.sync_copy(dst_hbm.at[wid], dst_sl)
        _zero_shared(zbuf, den_sh, sub, rows_sub, h)
        plsc.subcore_barrier()

        def issue(k, xlb, xrb, sl, sr):
            pltpu.async_copy(xlc_hbm.at[src_sl.at[k]], xlb, sl)
            pltpu.async_copy(xrc_hbm.at[dst_sl.at[k]], xrb, sr)

        def wait(k, xlb, xrb, sl, sr):
            pltpu.make_async_copy(xlc_hbm.at[src_sl.at[k]], xlb, sl).wait()
            pltpu.make_async_copy(xrc_hbm.at[dst_sl.at[k]], xrb, sr).wait()

        def compute(k, xlb, xrb):
            base = wid * epw + k * b

            def edge(ei, icarry):
                acc = jnp.zeros((L,), F32)
                for ci in range(c):
                    u = (xlb[ei, pl.ds(ci * L, L)]
                         + xrb[ei, pl.ds(ci * L, L)])
                    u = jnp.maximum(u, 0.2 * u)
                    acc = acc + u * attv[ci, :]
                expbuf[ei, :] = jnp.exp(acc)
                return icarry

            lax.fori_loop(0, b, edge, 0)
            pltpu.sync_copy(expbuf, expv_hbm.at[pl.ds(base, b)])
            pltpu.sync_copy(expbuf, den_sh.at[dst_sl.at[k]], add=True)

        issue(0, xl0, xr0, sl0, sr0)

        def pair(i, carry):
            k0 = 2 * i
            wait(k0, xl0, xr0, sl0, sr0)
            issue(k0 + 1, xl1, xr1, sl1, sr1)
            compute(k0, xl0, xr0)
            wait(k0 + 1, xl1, xr1, sl1, sr1)
            issue(k0 + 2, xl0, xr0, sl0, sr0)
            compute(k0 + 1, xl1, xr1)
            return carry

        lax.fori_loop(0, pairs, pair, 0)
        wait(chunks - 1, xl0, xr0, sl0, sr0)
        compute(chunks - 1, xl0, xr0)
        plsc.subcore_barrier()

        rows = pl.ds(sub * rows_sub, rows_sub)

        @pl.when(core == 0)
        def _():
            pltpu.sync_copy(den_sh.at[rows], den0_hbm.at[rows])

        @pl.when(core == 1)
        def _():
            pltpu.sync_copy(den_sh.at[rows], den1_hbm.at[rows])

    return p1(xlc, xrc, src3, dst3, att_t)


def _pass2(xlh, src3, dst3, expv, den0, den1, n, e, h, c, b, chunks):
    np_ = ((n + 8 * NS - 1) // (8 * NS)) * (8 * NS)
    rows_sub = np_ // NS
    mesh = plsc.VectorSubcoreMesh(core_axis_name="c", subcore_axis_name="s")
    epw = e // NW
    hc = h * c
    pairs = (chunks - 1) // 2

    @functools.partial(
        pl.kernel,
        out_type=[
            jax.ShapeDtypeStruct((e, h), F32),
            jax.ShapeDtypeStruct((np_, c), F32),
            jax.ShapeDtypeStruct((np_, c), F32),
        ],
        mesh=mesh,
        compiler_params=pltpu.CompilerParams(use_tc_tiling_on_sc=False),
        scratch_types=[
            pltpu.VMEM((chunks, b), I32),
            pltpu.VMEM((chunks, b), I32),
            pltpu.VMEM((b, hc), F32),
            pltpu.VMEM((b, hc), F32),
            pltpu.VMEM((b, h), F32),
            pltpu.VMEM((b, h), F32),
            pltpu.VMEM((b, h), F32),
            pltpu.VMEM((b, h), F32),
            pltpu.VMEM((b, h), F32),
            pltpu.VMEM((b, h), F32),
            pltpu.VMEM((b, h), F32),
            pltpu.VMEM((b, c), F32),
            pltpu.VMEM((rows_sub, c), F32),
            pltpu.VMEM_SHARED((np_, c), F32),
            pltpu.SemaphoreType.DMA,
            pltpu.SemaphoreType.DMA,
            pltpu.SemaphoreType.DMA,
            pltpu.SemaphoreType.DMA,
        ],
    )
    def p2(xlh_hbm, src_hbm, dst_hbm, expv_hbm, den0_hbm, den1_hbm,
           alpha_hbm, t0_hbm, t1_hbm,
           src_sl, dst_sl, xl0, xl1, ex0, ex1, d00, d01, d10, d11,
           albuf, msgbuf, zbuf, t_sh, sg0, sg1, ss0, ss1):
        core = lax.axis_index("c")
        sub = lax.axis_index("s")
        wid = core * NS + sub

        pltpu.sync_copy(src_hbm.at[wid], src_sl)
        pltpu.sync_copy(dst_hbm.at[wid], dst_sl)
        _zero_shared(zbuf, t_sh, sub, rows_sub, c)
        plsc.subcore_barrier()

        def issue(k, xlb, exb, d0b, d1b, sg, ss):
            base = wid * epw + k * b
            pltpu.async_copy(xlh_hbm.at[src_sl.at[k]], xlb, sg)
            pltpu.async_copy(expv_hbm.at[pl.ds(base, b)], exb, ss)
            pltpu.async_copy(den0_hbm.at[dst_sl.at[k]], d0b, ss)
            pltpu.async_copy(den1_hbm.at[dst_sl.at[k]], d1b, ss)

        def wait(k, xlb, exb, d0b, d1b, sg, ss):
            base = wid * epw + k * b
            pltpu.make_async_copy(xlh_hbm.at[src_sl.at[k]], xlb, sg).wait()
            pltpu.make_async_copy(expv_hbm.at[pl.ds(base, b)], exb, ss).wait()
            pltpu.make_async_copy(den0_hbm.at[dst_sl.at[k]], d0b, ss).wait()
            pltpu.make_async_copy(den1_hbm.at[dst_sl.at[k]], d1b, ss).wait()

        def compute(k, xlb, exb, d0b, d1b):
            base = wid * epw + k * b

            def edge(ei, icarry):
                den = d0b[ei, :] + d1b[ei, :]
                alpha_v = exb[ei, :] / (den + 1e-16)
                albuf[ei, :] = alpha_v
                for cg in range(c // L):
                    m = jnp.zeros((L,), F32)
                    for hi in range(h):
                        a_s = alpha_v[hi]
                        m = m + a_s * xlb[ei, pl.ds(hi * c + cg * L, L)]
                    msgbuf[ei, pl.ds(cg * L, L)] = m
                return icarry

            lax.fori_loop(0, b, edge, 0)
            pltpu.sync_copy(albuf, alpha_hbm.at[pl.ds(base, b)])
            pltpu.sync_copy(msgbuf, t_sh.at[dst_sl.at[k]], add=True)

        issue(0, xl0, ex0, d00, d10, sg0, ss0)

        def pair(i, carry):
            k0 = 2 * i
            wait(k0, xl0, ex0, d00, d10, sg0, ss0)
            issue(k0 + 1, xl1, ex1, d01, d11, sg1, ss1)
            compute(k0, xl0, ex0, d00, d10)
            wait(k0 + 1, xl1, ex1, d01, d11, sg1, ss1)
            issue(k0 + 2, xl0, ex0, d00, d10, sg0, ss0)
            compute(k0 + 1, xl1, ex1, d01, d11)
            return carry

        lax.fori_loop(0, pairs, pair, 0)
        wait(chunks - 1, xl0, ex0, d00, d10, sg0, ss0)
        compute(chunks - 1, xl0, ex0, d00, d10)
        plsc.subcore_barrier()

        rows = pl.ds(sub * rows_sub, rows_sub)

        @pl.when(core == 0)
        def _():
            pltpu.sync_copy(t_sh.at[rows], t0_hbm.at[rows])

        @pl.when(core == 1)
        def _():
            pltpu.sync_copy(t_sh.at[rows], t1_hbm.at[rows])

    return p2(xlh, src3, dst3, expv, den0, den1)


def _finalize(t0, t1, res, bias, h):
    n, oc = res.shape
    mb = 1000
    inv_h = 1.0 / h

    def body(t0_ref, t1_ref, res_ref, bias_ref, out_ref):
        g = (t0_ref[...] + t1_ref[...]) * inv_h + bias_ref[...]
        out_ref[...] = jnp.maximum(g, 0.0) + res_ref[...]

    return pl.pallas_call(
        body,
        grid=(n // mb,),
        in_specs=[
            pl.BlockSpec((mb, oc), lambda i: (i, 0)),
            pl.BlockSpec((mb, oc), lambda i: (i, 0)),
            pl.BlockSpec((mb, oc), lambda i: (i, 0)),
            pl.BlockSpec((1, oc), lambda i: (0, 0)),
        ],
        out_specs=pl.BlockSpec((mb, oc), lambda i: (i, 0)),
        out_shape=jax.ShapeDtypeStruct((n, oc), F32),
    )(t0, t1, res, bias.reshape(1, oc))


def kernel(x, edge_index, edge_attr, batch, W_l, W_r, att, bias, W_res):
    n, in_ch = x.shape
    h, c = att.shape
    e = edge_index.shape[1]
    b = 40
    chunks = e // (NW * b)

    src3 = edge_index[0].astype(I32).reshape(NW, chunks, b)
    dst3 = edge_index[1].astype(I32).reshape(NW, chunks, b)

    w_lc = W_l.reshape(in_ch, h, c).transpose(0, 2, 1).reshape(in_ch, h * c)
    w_rc = W_r.reshape(in_ch, h, c).transpose(0, 2, 1).reshape(in_ch, h * c)
    att_t = att.T.astype(F32)

    xlc, xrc, xlh, res = _matmuls(x, w_lc, w_rc, W_l, W_res)
    expv, den0, den1 = _pass1(xlc, xrc, src3, dst3, att_t, n, e, h, c, b,
                              chunks)
    alpha, t0, t1 = _pass2(xlh, src3, dst3, expv, den0, den1, n, e, h, c, b,
                           chunks)
    x_out = _finalize(t0[:n], t1[:n], res, bias, h)
    return (x_out, edge_index, edge_attr, batch, alpha)


# speedup vs baseline: 28.1985x; 1.0665x over previous
import functools

import jax
import jax.numpy as jnp
from jax import lax
from jax.experimental import pallas as pl
from jax.experimental.pallas import tpu as pltpu
from jax.experimental.pallas import tpu_sc as plsc

F32 = jnp.float32
I32 = jnp.int32

NC = 2
NS = 16
NW = NC * NS
L = 16
ZR = 64


def _matmuls(x, w_lc, w_rc, w_lh, w_res):
    n, k = x.shape
    hc = w_lc.shape[1]
    oc = w_res.shape[1]
    mb = 1000
    grid = n // mb

    def body(x_ref, wlc_ref, wrc_ref, wlh_ref, wres_ref, xlc_ref, xrc_ref,
             xlh_ref, res_ref):
        xb = x_ref[...]
        xlc_ref[...] = jnp.dot(xb, wlc_ref[...], preferred_element_type=F32)
        xrc_ref[...] = jnp.dot(xb, wrc_ref[...], preferred_element_type=F32)
        xlh_ref[...] = jnp.dot(xb, wlh_ref[...], preferred_element_type=F32)
        res_ref[...] = jnp.dot(xb, wres_ref[...], preferred_element_type=F32)

    return pl.pallas_call(
        body,
        grid=(grid,),
        in_specs=[
            pl.BlockSpec((mb, k), lambda i: (i, 0)),
            pl.BlockSpec((k, hc), lambda i: (0, 0)),
            pl.BlockSpec((k, hc), lambda i: (0, 0)),
            pl.BlockSpec((k, hc), lambda i: (0, 0)),
            pl.BlockSpec((k, oc), lambda i: (0, 0)),
        ],
        out_specs=[
            pl.BlockSpec((mb, hc), lambda i: (i, 0)),
            pl.BlockSpec((mb, hc), lambda i: (i, 0)),
            pl.BlockSpec((mb, hc), lambda i: (i, 0)),
            pl.BlockSpec((mb, oc), lambda i: (i, 0)),
        ],
        out_shape=[
            jax.ShapeDtypeStruct((n, hc), F32),
            jax.ShapeDtypeStruct((n, hc), F32),
            jax.ShapeDtypeStruct((n, hc), F32),
            jax.ShapeDtypeStruct((n, oc), F32),
        ],
    )(x, w_lc, w_rc, w_lh, w_res)


def _zero_shared(zbuf, shared, sub, rows_sub, width):
    zero = jnp.zeros((L,), F32)

    def zrow(i, carry):
        for g in range(width // L):
            zbuf[i, pl.ds(g * L, L)] = zero
        return carry

    lax.fori_loop(0, rows_sub, zrow, 0)
    pltpu.sync_copy(zbuf, shared.at[pl.ds(sub * rows_sub, rows_sub)])


def _pass1(xlc, xrc, src3, dst3, att_t, n, e, h, c, b, chunks):
    np_ = ((n + 8 * NS - 1) // (8 * NS)) * (8 * NS)
    rows_sub = np_ // NS
    mesh = plsc.VectorSubcoreMesh(core_axis_name="c", subcore_axis_name="s")
    epw = e // NW
    hc = h * c
    pairs = (chunks - 1) // 2

    @functools.partial(
        pl.kernel,
        out_type=[
            jax.ShapeDtypeStruct((e, h), F32),
            jax.ShapeDtypeStruct((np_, h), F32),
            jax.ShapeDtypeStruct((np_, h), F32),
        ],
        mesh=mesh,
        compiler_params=pltpu.CompilerParams(use_tc_tiling_on_sc=False),
        scratch_types=[
            pltpu.VMEM((chunks, b), I32),
            pltpu.VMEM((chunks, b), I32),
            pltpu.VMEM((b, hc), F32),
            pltpu.VMEM((b, hc), F32),
            pltpu.VMEM((b, hc), F32),
            pltpu.VMEM((b, hc), F32),
            pltpu.VMEM((b, h), F32),
            pltpu.VMEM((c, h), F32),
            pltpu.VMEM((rows_sub, h), F32),
            pltpu.VMEM_SHARED((np_, h), F32),
            pltpu.SemaphoreType.DMA,
            pltpu.SemaphoreType.DMA,
            pltpu.SemaphoreType.DMA,
            pltpu.SemaphoreType.DMA,
        ],
    )
    def p1(xlc_hbm, xrc_hbm, src_hbm, dst_hbm, att_hbm,
           expv_hbm, den0_hbm, den1_hbm,
           src_sl, dst_sl, xl0, xl1, xr0, xr1, expbuf, attv, zbuf, den_sh,
           sl0, sl1, sr0, sr1):
        core = lax.axis_index("c")
        sub = lax.axis_index("s")
        wid = core * NS + sub

        pltpu.sync_copy(att_hbm, attv)
        pltpu.sync_copy(src_hbm.at[wid], src_sl)
        pltpu.sync_copy(dst_hbm.at[wid], dst_sl)
        _zero_shared(zbuf, den_sh, sub, rows_sub, h)
        plsc.subcore_barrier()

        def issue(k, xlb, xrb, sl, sr):
            pltpu.async_copy(xlc_hbm.at[src_sl.at[k]], xlb, sl)
            pltpu.async_copy(xrc_hbm.at[dst_sl.at[k]], xrb, sr)

        def wait(k, xlb, xrb, sl, sr):
            pltpu.make_async_copy(xlc_hbm.at[src_sl.at[k]], xlb, sl).wait()
            pltpu.make_async_copy(xrc_hbm.at[dst_sl.at[k]], xrb, sr).wait()

        def compute(k, xlb, xrb):
            base = wid * epw + k * b

            def edge(ei, icarry):
                accs = [jnp.zeros((L,), F32) for _ in range(4)]
                for ci in range(c):
                    u = (xlb[ei, pl.ds(ci * L, L)]
                         + xrb[ei, pl.ds(ci * L, L)])
                    u = jnp.maximum(u, 0.2 * u)
                    accs[ci % 4] = accs[ci % 4] + u * attv[ci, :]
                acc = (accs[0] + accs[1]) + (accs[2] + accs[3])
                expbuf[ei, :] = jnp.exp(acc)
                return icarry

            lax.fori_loop(0, b, edge, 0)
            pltpu.sync_copy(expbuf, expv_hbm.at[pl.ds(base, b)])
            pltpu.sync_copy(expbuf, den_sh.at[dst_sl.at[k]], add=True)

        issue(0, xl0, xr0, sl0, sr0)

        def pair(i, carry):
            k0 = 2 * i
            wait(k0, xl0, xr0, sl0, sr0)
            issue(k0 + 1, xl1, xr1, sl1, sr1)
            compute(k0, xl0, xr0)
            wait(k0 + 1, xl1, xr1, sl1, sr1)
            issue(k0 + 2, xl0, xr0, sl0, sr0)
            compute(k0 + 1, xl1, xr1)
            return carry

        lax.fori_loop(0, pairs, pair, 0)
        wait(chunks - 1, xl0, xr0, sl0, sr0)
        compute(chunks - 1, xl0, xr0)
        plsc.subcore_barrier()

        rows = pl.ds(sub * rows_sub, rows_sub)

        @pl.when(core == 0)
        def _():
            pltpu.sync_copy(den_sh.at[rows], den0_hbm.at[rows])

        @pl.when(core == 1)
        def _():
            pltpu.sync_copy(den_sh.at[rows], den1_hbm.at[rows])

    return p1(xlc, xrc, src3, dst3, att_t)


def _pass2(xlh, src3, dst3, expv, den0, den1, n, e, h, c, b, chunks):
    np_ = ((n + 8 * NS - 1) // (8 * NS)) * (8 * NS)
    rows_sub = np_ // NS
    mesh = plsc.VectorSubcoreMesh(core_axis_name="c", subcore_axis_name="s")
    epw = e // NW
    hc = h * c
    pairs = (chunks - 1) // 2

    @functools.partial(
        pl.kernel,
        out_type=[
            jax.ShapeDtypeStruct((e, h), F32),
            jax.ShapeDtypeStruct((np_, c), F32),
            jax.ShapeDtypeStruct((np_, c), F32),
        ],
        mesh=mesh,
        compiler_params=pltpu.CompilerParams(use_tc_tiling_on_sc=False),
        scratch_types=[
            pltpu.VMEM((chunks, b), I32),
            pltpu.VMEM((chunks, b), I32),
            pltpu.VMEM((b, hc), F32),
            pltpu.VMEM((b, hc), F32),
            pltpu.VMEM((b, h), F32),
            pltpu.VMEM((b, h), F32),
            pltpu.VMEM((b, h), F32),
            pltpu.VMEM((b, h), F32),
            pltpu.VMEM((b, h), F32),
            pltpu.VMEM((b, h), F32),
            pltpu.VMEM((b, h), F32),
            pltpu.VMEM((b, c), F32),
            pltpu.VMEM((rows_sub, c), F32),
            pltpu.VMEM_SHARED((np_, c), F32),
            pltpu.SemaphoreType.DMA,
            pltpu.SemaphoreType.DMA,
            pltpu.SemaphoreType.DMA,
            pltpu.SemaphoreType.DMA,
        ],
    )
    def p2(xlh_hbm, src_hbm, dst_hbm, expv_hbm, den0_hbm, den1_hbm,
           alpha_hbm, t0_hbm, t1_hbm,
           src_sl, dst_sl, xl0, xl1, ex0, ex1, d00, d01, d10, d11,
           albuf, msgbuf, zbuf, t_sh, sg0, sg1, ss0, ss1):
        core = lax.axis_index("c")
        sub = lax.axis_index("s")
        wid = core * NS + sub

        pltpu.sync_copy(src_hbm.at[wid], src_sl)
        pltpu.sync_copy(dst_hbm.at[wid], dst_sl)
        _zero_shared(zbuf, t_sh, sub, rows_sub, c)
        plsc.subcore_barrier()

        def issue(k, xlb, exb, d0b, d1b, sg, ss):
            base = wid * epw + k * b
            pltpu.async_copy(xlh_hbm.at[src_sl.at[k]], xlb, sg)
            pltpu.async_copy(expv_hbm.at[pl.ds(base, b)], exb, ss)
            pltpu.async_copy(den0_hbm.at[dst_sl.at[k]], d0b, ss)
            pltpu.async_copy(den1_hbm.at[dst_sl.at[k]], d1b, ss)

        def wait(k, xlb, exb, d0b, d1b, sg, ss):
            base = wid * epw + k * b
            pltpu.make_async_copy(xlh_hbm.at[src_sl.at[k]], xlb, sg).wait()
            pltpu.make_async_copy(expv_hbm.at[pl.ds(base, b)], exb, ss).wait()
            pltpu.make_async_copy(den0_hbm.at[dst_sl.at[k]], d0b, ss).wait()
            pltpu.make_async_copy(den1_hbm.at[dst_sl.at[k]], d1b, ss).wait()

        def compute(k, xlb, exb, d0b, d1b):
            base = wid * epw + k * b

            def edge(ei, icarry):
                den = d0b[ei, :] + d1b[ei, :]
                alpha_v = exb[ei, :] / (den + 1e-16)
                albuf[ei, :] = alpha_v
                for cg in range(c // L):
                    m0 = jnp.zeros((L,), F32)
                    m1 = jnp.zeros((L,), F32)
                    m2 = jnp.zeros((L,), F32)
                    m3 = jnp.zeros((L,), F32)
                    for hi in range(0, h, 4):
                        m0 = m0 + alpha_v[hi] * xlb[ei, pl.ds(hi * c + cg * L, L)]
                        m1 = m1 + alpha_v[hi + 1] * xlb[ei, pl.ds((hi + 1) * c + cg * L, L)]
                        m2 = m2 + alpha_v[hi + 2] * xlb[ei, pl.ds((hi + 2) * c + cg * L, L)]
                        m3 = m3 + alpha_v[hi + 3] * xlb[ei, pl.ds((hi + 3) * c + cg * L, L)]
                    msgbuf[ei, pl.ds(cg * L, L)] = (m0 + m1) + (m2 + m3)
                return icarry

            lax.fori_loop(0, b, edge, 0)
            pltpu.sync_copy(albuf, alpha_hbm.at[pl.ds(base, b)])
            pltpu.sync_copy(msgbuf, t_sh.at[dst_sl.at[k]], add=True)

        issue(0, xl0, ex0, d00, d10, sg0, ss0)

        def pair(i, carry):
            k0 = 2 * i
            wait(k0, xl0, ex0, d00, d10, sg0, ss0)
            issue(k0 + 1, xl1, ex1, d01, d11, sg1, ss1)
            compute(k0, xl0, ex0, d00, d10)
            wait(k0 + 1, xl1, ex1, d01, d11, sg1, ss1)
            issue(k0 + 2, xl0, ex0, d00, d10, sg0, ss0)
            compute(k0 + 1, xl1, ex1, d01, d11)
            return carry

        lax.fori_loop(0, pairs, pair, 0)
        wait(chunks - 1, xl0, ex0, d00, d10, sg0, ss0)
        compute(chunks - 1, xl0, ex0, d00, d10)
        plsc.subcore_barrier()

        rows = pl.ds(sub * rows_sub, rows_sub)

        @pl.when(core == 0)
        def _():
            pltpu.sync_copy(t_sh.at[rows], t0_hbm.at[rows])

        @pl.when(core == 1)
        def _():
            pltpu.sync_copy(t_sh.at[rows], t1_hbm.at[rows])

    return p2(xlh, src3, dst3, expv, den0, den1)


def _finalize(t0, t1, res, bias, h):
    n, oc = res.shape
    mb = 1000
    inv_h = 1.0 / h

    def body(t0_ref, t1_ref, res_ref, bias_ref, out_ref):
        g = (t0_ref[...] + t1_ref[...]) * inv_h + bias_ref[...]
        out_ref[...] = jnp.maximum(g, 0.0) + res_ref[...]

    return pl.pallas_call(
        body,
        grid=(n // mb,),
        in_specs=[
            pl.BlockSpec((mb, oc), lambda i: (i, 0)),
            pl.BlockSpec((mb, oc), lambda i: (i, 0)),
            pl.BlockSpec((mb, oc), lambda i: (i, 0)),
            pl.BlockSpec((1, oc), lambda i: (0, 0)),
        ],
        out_specs=pl.BlockSpec((mb, oc), lambda i: (i, 0)),
        out_shape=jax.ShapeDtypeStruct((n, oc), F32),
    )(t0, t1, res, bias.reshape(1, oc))


def kernel(x, edge_index, edge_attr, batch, W_l, W_r, att, bias, W_res):
    n, in_ch = x.shape
    h, c = att.shape
    e = edge_index.shape[1]
    b = 40
    chunks = e // (NW * b)

    src3 = edge_index[0].astype(I32).reshape(NW, chunks, b)
    dst3 = edge_index[1].astype(I32).reshape(NW, chunks, b)

    w_lc = W_l.reshape(in_ch, h, c).transpose(0, 2, 1).reshape(in_ch, h * c)
    w_rc = W_r.reshape(in_ch, h, c).transpose(0, 2, 1).reshape(in_ch, h * c)
    att_t = att.T.astype(F32)

    xlc, xrc, xlh, res = _matmuls(x, w_lc, w_rc, W_l, W_res)
    expv, den0, den1 = _pass1(xlc, xrc, src3, dst3, att_t, n, e, h, c, b,
                              chunks)
    alpha, t0, t1 = _pass2(xlh, src3, dst3, expv, den0, den1, n, e, h, c, b,
                           chunks)
    x_out = _finalize(t0, t1, res, bias, h)
    return (x_out, edge_index, edge_attr, batch, alpha)
